# baseline (speedup 1.0000x reference)
import functools

import jax
import jax.numpy as jnp
from jax import lax
from jax.experimental import pallas as pl
from jax.experimental.pallas import tpu as pltpu
from jax.experimental.pallas import tpu_sc as plsc

_E = 8
_K = 2
_H = 1024
_I = 2816
_T = 2048
_B = 256
_NBLK = _T * _K // _B + _E
_NBLK_PAD = 32
_PAD = _NBLK * _B
_J = 2
_IJ = _I // _J

_W = 128
_NW = 32
_TC = _T // _NW


def _route_body(rl_ref, p0_ref, p1_ref, w0_ref, w1_ref, be_ref, vd_ref):
    logits = rl_ref[...]
    probs = jax.nn.softmax(logits, axis=-1)
    ids = lax.broadcasted_iota(jnp.int32, probs.shape, 1)
    top1 = jnp.max(probs, axis=-1, keepdims=True)
    i1 = jnp.min(jnp.where(probs == top1, ids, _E), axis=-1, keepdims=True)
    masked = jnp.where(ids == i1, -jnp.inf, probs)
    top2 = jnp.max(masked, axis=-1, keepdims=True)
    i2 = jnp.min(jnp.where(masked == top2, ids, _E), axis=-1, keepdims=True)
    denom = top1 + top2

    sel = ((ids == i1) | (ids == i2)).astype(jnp.int32)
    cum = sel
    s = 1
    while s < _T:
        shifted = jnp.concatenate(
            [jnp.zeros((s, _E), jnp.int32), cum[: _T - s, :]], axis=0)
        cum = cum + shifted
        s *= 2
    rank = cum - sel
    cnt = cum[_T - 1:_T, :]
    cpad = ((cnt + (_B - 1)) // _B) * _B
    offi = cpad
    s = 1
    while s < _E:
        shifted = jnp.concatenate(
            [jnp.zeros((1, s), jnp.int32), offi[:, : _E - s]], axis=1)
        offi = offi + shifted
        s *= 2
    off = offi - cpad

    pos_all = off + rank
    p0 = jnp.sum(jnp.where(ids == i1, pos_all, 0), axis=-1, keepdims=True)
    p1 = jnp.sum(jnp.where(ids == i2, pos_all, 0), axis=-1, keepdims=True)
    p0_ref[...] = p0
    p1_ref[...] = p1
    w0_ref[...] = jnp.broadcast_to(top1 / denom, (_T, _W))
    w1_ref[...] = jnp.broadcast_to(top2 / denom, (_T, _W))

    bl = lax.broadcasted_iota(jnp.int32, (1, _NBLK_PAD), 1) * _B
    be = jnp.zeros((1, _NBLK_PAD), jnp.int32)
    for e in range(_E):
        be = be + (bl >= off[0, e] + cpad[0, e]).astype(jnp.int32)
    vd = jnp.zeros((1, _NBLK_PAD), jnp.int32)
    for e in range(_E):
        vd = vd + jnp.where(
            be == e, jnp.clip(off[0, e] + cnt[0, e] - bl, 0, _B), 0)
    be_ref[...] = jnp.minimum(be, _E - 1)
    nblk_used = (off[0, _E - 1] + cpad[0, _E - 1]) // _B
    lane = lax.broadcasted_iota(jnp.int32, (1, _NBLK_PAD), 1)
    vd_ref[...] = vd + jnp.where(lane == _NBLK_PAD - 1, nblk_used, 0)


def _route(router_logits):
    return pl.pallas_call(
        _route_body,
        in_specs=[pl.BlockSpec((_T, _E), lambda: (0, 0))],
        out_specs=[
            pl.BlockSpec((_T, 1), lambda: (0, 0)),
            pl.BlockSpec((_T, 1), lambda: (0, 0)),
            pl.BlockSpec((_T, _W), lambda: (0, 0)),
            pl.BlockSpec((_T, _W), lambda: (0, 0)),
            pl.BlockSpec((1, _NBLK_PAD), lambda: (0, 0)),
            pl.BlockSpec((1, _NBLK_PAD), lambda: (0, 0)),
        ],
        out_shape=[
            jax.ShapeDtypeStruct((_T, 1), jnp.int32),
            jax.ShapeDtypeStruct((_T, 1), jnp.int32),
            jax.ShapeDtypeStruct((_T, _W), jnp.float32),
            jax.ShapeDtypeStruct((_T, _W), jnp.float32),
            jax.ShapeDtypeStruct((1, _NBLK_PAD), jnp.int32),
            jax.ShapeDtypeStruct((1, _NBLK_PAD), jnp.int32),
        ],
    )(router_logits)


def _sc_mesh():
    return plsc.VectorSubcoreMesh(core_axis_name="c", subcore_axis_name="s")


def _scatter_body(x_hbm, p0_hbm, p1_hbm, w0_hbm, w1_hbm, xs_hbm, ws_hbm,
                  xv, i0, i1, wv0, wv1, sem):
    wid = lax.axis_index("s") * 2 + lax.axis_index("c")
    base = wid * _TC
    pltpu.sync_copy(p0_hbm.at[pl.ds(base, _TC)], i0)
    pltpu.sync_copy(p1_hbm.at[pl.ds(base, _TC)], i1)
    pltpu.sync_copy(x_hbm.at[pl.ds(base, _TC)], xv)
    pltpu.sync_copy(w0_hbm.at[pl.ds(base, _TC)], wv0)
    pltpu.sync_copy(w1_hbm.at[pl.ds(base, _TC)], wv1)
    c0 = pltpu.async_copy(xv, xs_hbm.at[i0], sem)
    c1 = pltpu.async_copy(xv, xs_hbm.at[i1], sem)
    c2 = pltpu.async_copy(wv0, ws_hbm.at[i0], sem)
    c3 = pltpu.async_copy(wv1, ws_hbm.at[i1], sem)
    c0.wait()
    c1.wait()
    c2.wait()
    c3.wait()


def _scatter(x, p0, p1, w0, w1):
    f = pl.kernel(
        _scatter_body,
        mesh=_sc_mesh(),
        out_type=[
            jax.ShapeDtypeStruct((_PAD, _H), jnp.float32),
            jax.ShapeDtypeStruct((_PAD, _W), jnp.float32),
        ],
        scratch_types=[
            pltpu.VMEM((_TC, _H), jnp.float32),
            pltpu.VMEM((_TC,), jnp.int32),
            pltpu.VMEM((_TC,), jnp.int32),
            pltpu.VMEM((_TC, _W), jnp.float32),
            pltpu.VMEM((_TC, _W), jnp.float32),
            pltpu.SemaphoreType.DMA,
        ],
    )
    return f(x, p0, p1, w0, w1)


def _ieff(ii, vd):
    return jnp.minimum(ii, vd[0, _NBLK_PAD - 1] - 1)


def _im_rows(ii, j, be, vd):
    return (_ieff(ii, vd), 0)


def _jj(i, j):
    return jnp.where(i % 2 == 1, _J - 1 - j, j)


def _im_gu(ii, j, be, vd):
    i = _ieff(ii, vd)
    return (be[0, i], 0, _jj(i, j))


def _im_d(ii, j, be, vd):
    i = _ieff(ii, vd)
    return (be[0, i], _jj(i, j), 0)


def _gmm_body(be_ref, vd_ref, xs_ref, ws_ref, g_ref, u_ref, d_ref, ys_ref):
    ii = pl.program_id(0)
    j = pl.program_id(1)

    @pl.when(ii < vd_ref[0, _NBLK_PAD - 1])
    def _():
        xs16 = xs_ref[...].astype(jnp.bfloat16)
        g16 = g_ref[0].astype(jnp.bfloat16)
        u16 = u_ref[0].astype(jnp.bfloat16)
        d16 = d_ref[0].astype(jnp.bfloat16)
        gg = jnp.dot(xs16, g16, preferred_element_type=jnp.float32)
        uu = jnp.dot(xs16, u16, preferred_element_type=jnp.float32)
        w = ws_ref[...][:, :1]
        h = ((w * (gg * jax.lax.logistic(gg))) * uu).astype(jnp.bfloat16)
        yw = jnp.dot(h, d16, preferred_element_type=jnp.float32)

        @pl.when(j == 0)
        def _():
            ys_ref[...] = yw

        @pl.when(j != 0)
        def _():
            ys_ref[...] = ys_ref[...] + yw


def _gmm(xs, ws, be, vd, gate_proj, up_proj, down_proj):
    grid_spec = pltpu.PrefetchScalarGridSpec(
        num_scalar_prefetch=2,
        grid=(_NBLK, _J),
        in_specs=[
            pl.BlockSpec((_B, _H), _im_rows),
            pl.BlockSpec((_B, _W), _im_rows),
            pl.BlockSpec((1, _H, _IJ), _im_gu),
            pl.BlockSpec((1, _H, _IJ), _im_gu),
            pl.BlockSpec((1, _IJ, _H), _im_d),
        ],
        out_specs=pl.BlockSpec((_B, _H), _im_rows),
    )
    return pl.pallas_call(
        _gmm_body,
        grid_spec=grid_spec,
        out_shape=jax.ShapeDtypeStruct((_PAD, _H), jnp.float32),
        compiler_params=pltpu.CompilerParams(
            dimension_semantics=("arbitrary", "arbitrary"),
        ),
    )(be, vd, xs, ws, gate_proj, up_proj, down_proj)


def _combine_body(ys_hbm, p0_hbm, p1_hbm, out_hbm, i0, i1, y0, y1, sem):
    wid = lax.axis_index("s") * 2 + lax.axis_index("c")
    for half in range(2):
        b = wid * _TC + half * (_TC // 2)
        pltpu.sync_copy(p0_hbm.at[pl.ds(b, _TC // 2)], i0)
        pltpu.sync_copy(p1_hbm.at[pl.ds(b, _TC // 2)], i1)
        c0 = pltpu.async_copy(ys_hbm.at[i0], y0, sem)
        c1 = pltpu.async_copy(ys_hbm.at[i1], y1, sem)
        c0.wait()
        c1.wait()

        def tok(t, _):
            for col in range(0, _H, 16):
                cs = pl.ds(col, 16)
                y0[t, cs] = y0[t, cs] + y1[t, cs]
            return 0

        lax.fori_loop(0, _TC // 2, tok, 0)
        pltpu.sync_copy(y0, out_hbm.at[pl.ds(b, _TC // 2)])


def _combine(ys, p0, p1):
    f = pl.kernel(
        _combine_body,
        mesh=_sc_mesh(),
        out_type=jax.ShapeDtypeStruct((_T, _H), jnp.float32),
        scratch_types=[
            pltpu.VMEM((_TC // 2,), jnp.int32),
            pltpu.VMEM((_TC // 2,), jnp.int32),
            pltpu.VMEM((_TC // 2, _H), jnp.float32),
            pltpu.VMEM((_TC // 2, _H), jnp.float32),
            pltpu.SemaphoreType.DMA,
        ],
    )
    return f(ys, p0, p1)


def kernel(x, router_logits, gate_proj, up_proj, down_proj):
    p0, p1, w0, w1, be, vd = _route(router_logits)
    p0 = jnp.reshape(p0, (_T,))
    p1 = jnp.reshape(p1, (_T,))
    xs, ws = _scatter(x, p0, p1, w0, w1)
    ys = _gmm(xs, ws, be, vd, gate_proj, up_proj, down_proj)
    return _combine(ys, p0, p1)


# speedup vs baseline: 1.2035x; 1.2035x over previous
import functools

import jax
import jax.numpy as jnp
from jax import lax
from jax.experimental import pallas as pl
from jax.experimental.pallas import tpu as pltpu
from jax.experimental.pallas import tpu_sc as plsc

_E = 8
_K = 2
_H = 1024
_I = 2816
_T = 2048
_B = 512
_NBLK = _T * _K // _B + _E
_NBLK_PAD = 32
_PAD = _NBLK * _B
_J = 2
_IJ = _I // _J

_W = 128
_NW = 32
_TC = _T // _NW


def _route_body(rl_ref, p0_ref, p1_ref, w0_ref, w1_ref, be_ref, vd_ref):
    logits = rl_ref[...]
    probs = jax.nn.softmax(logits, axis=-1)
    ids = lax.broadcasted_iota(jnp.int32, probs.shape, 1)
    top1 = jnp.max(probs, axis=-1, keepdims=True)
    i1 = jnp.min(jnp.where(probs == top1, ids, _E), axis=-1, keepdims=True)
    masked = jnp.where(ids == i1, -jnp.inf, probs)
    top2 = jnp.max(masked, axis=-1, keepdims=True)
    i2 = jnp.min(jnp.where(masked == top2, ids, _E), axis=-1, keepdims=True)
    denom = top1 + top2

    sel = ((ids == i1) | (ids == i2)).astype(jnp.int32)
    cum = sel
    s = 1
    while s < _T:
        shifted = jnp.concatenate(
            [jnp.zeros((s, _E), jnp.int32), cum[: _T - s, :]], axis=0)
        cum = cum + shifted
        s *= 2
    rank = cum - sel
    cnt = cum[_T - 1:_T, :]
    cpad = ((cnt + (_B - 1)) // _B) * _B
    offi = cpad
    s = 1
    while s < _E:
        shifted = jnp.concatenate(
            [jnp.zeros((1, s), jnp.int32), offi[:, : _E - s]], axis=1)
        offi = offi + shifted
        s *= 2
    off = offi - cpad

    pos_all = off + rank
    p0 = jnp.sum(jnp.where(ids == i1, pos_all, 0), axis=-1, keepdims=True)
    p1 = jnp.sum(jnp.where(ids == i2, pos_all, 0), axis=-1, keepdims=True)
    p0_ref[...] = p0
    p1_ref[...] = p1
    w0_ref[...] = jnp.broadcast_to(top1 / denom, (_T, _W))
    w1_ref[...] = jnp.broadcast_to(top2 / denom, (_T, _W))

    bl = lax.broadcasted_iota(jnp.int32, (1, _NBLK_PAD), 1) * _B
    be = jnp.zeros((1, _NBLK_PAD), jnp.int32)
    for e in range(_E):
        be = be + (bl >= off[0, e] + cpad[0, e]).astype(jnp.int32)
    vd = jnp.zeros((1, _NBLK_PAD), jnp.int32)
    for e in range(_E):
        vd = vd + jnp.where(
            be == e, jnp.clip(off[0, e] + cnt[0, e] - bl, 0, _B), 0)
    be_ref[...] = jnp.minimum(be, _E - 1)
    nblk_used = (off[0, _E - 1] + cpad[0, _E - 1]) // _B
    lane = lax.broadcasted_iota(jnp.int32, (1, _NBLK_PAD), 1)
    vd_ref[...] = vd + jnp.where(lane == _NBLK_PAD - 1, nblk_used, 0)


def _route(router_logits):
    return pl.pallas_call(
        _route_body,
        in_specs=[pl.BlockSpec((_T, _E), lambda: (0, 0))],
        out_specs=[
            pl.BlockSpec((_T, 1), lambda: (0, 0)),
            pl.BlockSpec((_T, 1), lambda: (0, 0)),
            pl.BlockSpec((_T, _W), lambda: (0, 0)),
            pl.BlockSpec((_T, _W), lambda: (0, 0)),
            pl.BlockSpec((1, _NBLK_PAD), lambda: (0, 0)),
            pl.BlockSpec((1, _NBLK_PAD), lambda: (0, 0)),
        ],
        out_shape=[
            jax.ShapeDtypeStruct((_T, 1), jnp.int32),
            jax.ShapeDtypeStruct((_T, 1), jnp.int32),
            jax.ShapeDtypeStruct((_T, _W), jnp.float32),
            jax.ShapeDtypeStruct((_T, _W), jnp.float32),
            jax.ShapeDtypeStruct((1, _NBLK_PAD), jnp.int32),
            jax.ShapeDtypeStruct((1, _NBLK_PAD), jnp.int32),
        ],
    )(router_logits)


def _sc_mesh():
    return plsc.VectorSubcoreMesh(core_axis_name="c", subcore_axis_name="s")


def _scatter_body(x_hbm, p0_hbm, p1_hbm, w0_hbm, w1_hbm, xs_hbm, ws_hbm,
                  xv, i0, i1, wv0, wv1, sem):
    wid = lax.axis_index("s") * 2 + lax.axis_index("c")
    base = wid * _TC
    pltpu.sync_copy(p0_hbm.at[pl.ds(base, _TC)], i0)
    pltpu.sync_copy(p1_hbm.at[pl.ds(base, _TC)], i1)
    pltpu.sync_copy(x_hbm.at[pl.ds(base, _TC)], xv)
    pltpu.sync_copy(w0_hbm.at[pl.ds(base, _TC)], wv0)
    pltpu.sync_copy(w1_hbm.at[pl.ds(base, _TC)], wv1)
    c0 = pltpu.async_copy(xv, xs_hbm.at[i0], sem)
    c1 = pltpu.async_copy(xv, xs_hbm.at[i1], sem)
    c2 = pltpu.async_copy(wv0, ws_hbm.at[i0], sem)
    c3 = pltpu.async_copy(wv1, ws_hbm.at[i1], sem)
    c0.wait()
    c1.wait()
    c2.wait()
    c3.wait()


def _scatter(x, p0, p1, w0, w1):
    f = pl.kernel(
        _scatter_body,
        mesh=_sc_mesh(),
        out_type=[
            jax.ShapeDtypeStruct((_PAD, _H), jnp.float32),
            jax.ShapeDtypeStruct((_PAD, _W), jnp.float32),
        ],
        scratch_types=[
            pltpu.VMEM((_TC, _H), jnp.float32),
            pltpu.VMEM((_TC,), jnp.int32),
            pltpu.VMEM((_TC,), jnp.int32),
            pltpu.VMEM((_TC, _W), jnp.float32),
            pltpu.VMEM((_TC, _W), jnp.float32),
            pltpu.SemaphoreType.DMA,
        ],
    )
    return f(x, p0, p1, w0, w1)


def _ieff(ii, vd):
    return jnp.minimum(ii, vd[0, _NBLK_PAD - 1] - 1)


def _im_rows(ii, j, be, vd):
    return (_ieff(ii, vd), 0)


def _jj(i, j):
    return jnp.where(i % 2 == 1, _J - 1 - j, j)


def _im_gu(ii, j, be, vd):
    i = _ieff(ii, vd)
    return (be[0, i], 0, _jj(i, j))


def _im_d(ii, j, be, vd):
    i = _ieff(ii, vd)
    return (be[0, i], _jj(i, j), 0)


def _gmm_body(be_ref, vd_ref, xs_ref, ws_ref, g_ref, u_ref, d_ref, ys_ref):
    ii = pl.program_id(0)
    j = pl.program_id(1)

    @pl.when(ii < vd_ref[0, _NBLK_PAD - 1])
    def _():
        xs16 = xs_ref[...].astype(jnp.bfloat16)
        g16 = g_ref[0].astype(jnp.bfloat16)
        u16 = u_ref[0].astype(jnp.bfloat16)
        d16 = d_ref[0].astype(jnp.bfloat16)
        gg = jnp.dot(xs16, g16, preferred_element_type=jnp.float32)
        uu = jnp.dot(xs16, u16, preferred_element_type=jnp.float32)
        w = ws_ref[...][:, :1]
        h = ((w * (gg * jax.lax.logistic(gg))) * uu).astype(jnp.bfloat16)
        yw = jnp.dot(h, d16, preferred_element_type=jnp.float32)

        @pl.when(j == 0)
        def _():
            ys_ref[...] = yw

        @pl.when(j != 0)
        def _():
            ys_ref[...] = ys_ref[...] + yw


def _gmm(xs, ws, be, vd, gate_proj, up_proj, down_proj):
    grid_spec = pltpu.PrefetchScalarGridSpec(
        num_scalar_prefetch=2,
        grid=(_NBLK, _J),
        in_specs=[
            pl.BlockSpec((_B, _H), _im_rows),
            pl.BlockSpec((_B, _W), _im_rows),
            pl.BlockSpec((1, _H, _IJ), _im_gu),
            pl.BlockSpec((1, _H, _IJ), _im_gu),
            pl.BlockSpec((1, _IJ, _H), _im_d),
        ],
        out_specs=pl.BlockSpec((_B, _H), _im_rows),
    )
    return pl.pallas_call(
        _gmm_body,
        grid_spec=grid_spec,
        out_shape=jax.ShapeDtypeStruct((_PAD, _H), jnp.float32),
        compiler_params=pltpu.CompilerParams(
            dimension_semantics=("arbitrary", "arbitrary"),
        ),
    )(be, vd, xs, ws, gate_proj, up_proj, down_proj)


def _combine_body(ys_hbm, p0_hbm, p1_hbm, out_hbm, i0, i1, y0, y1, sem):
    wid = lax.axis_index("s") * 2 + lax.axis_index("c")
    for half in range(2):
        b = wid * _TC + half * (_TC // 2)
        pltpu.sync_copy(p0_hbm.at[pl.ds(b, _TC // 2)], i0)
        pltpu.sync_copy(p1_hbm.at[pl.ds(b, _TC // 2)], i1)
        c0 = pltpu.async_copy(ys_hbm.at[i0], y0, sem)
        c1 = pltpu.async_copy(ys_hbm.at[i1], y1, sem)
        c0.wait()
        c1.wait()

        def tok(t, _):
            for col in range(0, _H, 16):
                cs = pl.ds(col, 16)
                y0[t, cs] = y0[t, cs] + y1[t, cs]
            return 0

        lax.fori_loop(0, _TC // 2, tok, 0)
        pltpu.sync_copy(y0, out_hbm.at[pl.ds(b, _TC // 2)])


def _combine(ys, p0, p1):
    f = pl.kernel(
        _combine_body,
        mesh=_sc_mesh(),
        out_type=jax.ShapeDtypeStruct((_T, _H), jnp.float32),
        scratch_types=[
            pltpu.VMEM((_TC // 2,), jnp.int32),
            pltpu.VMEM((_TC // 2,), jnp.int32),
            pltpu.VMEM((_TC // 2, _H), jnp.float32),
            pltpu.VMEM((_TC // 2, _H), jnp.float32),
            pltpu.SemaphoreType.DMA,
        ],
    )
    return f(ys, p0, p1)


def kernel(x, router_logits, gate_proj, up_proj, down_proj):
    p0, p1, w0, w1, be, vd = _route(router_logits)
    p0 = jnp.reshape(p0, (_T,))
    p1 = jnp.reshape(p1, (_T,))
    xs, ws = _scatter(x, p0, p1, w0, w1)
    ys = _gmm(xs, ws, be, vd, gate_proj, up_proj, down_proj)
    return _combine(ys, p0, p1)
